# baseline (speedup 1.0000x reference)
import functools

import jax
import jax.numpy as jnp
from jax import lax
from jax.experimental import pallas as pl
from jax.experimental.pallas import tpu as pltpu
from jax.experimental.pallas import tpu_sc as plsc

LAT_MIN, LAT_MAX = -90.0, 90.0
LON_MIN, LON_MAX = -180.0, 180.0

NC = 2
NS = 16
NW = NC * NS
T = 1024
GW = 128
GPC = T // GW


@functools.lru_cache(maxsize=None)
def _build(N):
    R = N // NW
    n_chunks = R // T
    pairs = n_chunks // 2

    mesh = plsc.VectorSubcoreMesh(core_axis_name="c", subcore_axis_name="s")

    @functools.partial(
        pl.kernel,
        mesh=mesh,
        out_type=jax.ShapeDtypeStruct((N, 16), jnp.float32),
        compiler_params=pltpu.CompilerParams(
            needs_layout_passes=False, use_tc_tiling_on_sc=False),
        scratch_types=[
            pltpu.VMEM((2, GPC, GW), jnp.int32),
            pltpu.VMEM((2, 2 * T), jnp.float32),
            pltpu.VMEM((2, T, 16), jnp.float32),
            pltpu.VMEM((18, 16), jnp.float32),
            [pltpu.SemaphoreType.DMA] * 6,
        ],
    )
    def k(tab_hbm, idx_hbm, loc_hbm, const_hbm, out_hbm,
          idx_v, loc_v, out_v, const_v, sems):
        isem, gsem, osem = sems[0:2], sems[2:4], sems[4:6]
        wid = lax.axis_index("s") * NC + lax.axis_index("c")
        base0 = wid * R
        ibase0 = wid * (R // GW)
        pltpu.sync_copy(const_hbm, const_v)
        iota = lax.iota(jnp.int32, 16)
        ca = [const_v[3 * c] for c in range(6)]
        cb = [const_v[3 * c + 1] for c in range(6)]
        cc = [const_v[3 * c + 2] for c in range(6)]
        cols = [jnp.full((16,), c, jnp.int32) for c in range(6)]
        evens = iota * 2

        def in_start(c, b):
            pltpu.async_copy(idx_hbm.at[pl.ds(ibase0 + c * GPC, GPC)],
                             idx_v.at[b], isem[b])
            pltpu.async_copy(loc_hbm.at[pl.ds(2 * (base0 + c * T), 2 * T)],
                             loc_v.at[b], isem[b])

        def in_wait(b):
            pltpu.make_async_copy(idx_hbm.at[pl.ds(ibase0, GPC)],
                                  idx_v.at[b], isem[b]).wait()
            pltpu.make_async_copy(loc_hbm.at[pl.ds(0, 2 * T)],
                                  loc_v.at[b], isem[b]).wait()

        def gather_start(b):
            for j in range(GPC):
                pltpu.async_copy(tab_hbm.at[idx_v.at[b, j]],
                                 out_v.at[b, pl.ds(j * GW, GW)], gsem[b])

        def gather_wait(b):
            for j in range(GPC):
                pltpu.make_async_copy(tab_hbm.at[idx_v.at[b, j]],
                                      out_v.at[b, pl.ds(j * GW, GW)],
                                      gsem[b]).wait()

        def out_start(c, b):
            pltpu.async_copy(out_v.at[b],
                             out_hbm.at[pl.ds(base0 + c * T, T)], osem[b])

        def out_wait(b):
            pltpu.make_async_copy(out_v.at[b],
                                  out_hbm.at[pl.ds(base0, T)], osem[b]).wait()

        def compute(b):
            def group_body(g, carry):
                li = g * 32 + evens
                lat = plsc.load_gather(loc_v.at[b], [li])
                lon = plsc.load_gather(loc_v.at[b], [li + 1])
                rows = g * 16 + iota
                for c in range(6):
                    vals = lat * ca[c] + lon * cb[c] + cc[c]
                    plsc.store_scatter(out_v.at[b], [rows, cols[c]], vals)
                return carry
            lax.fori_loop(0, T // 16, group_body, 0)

        def half(c, b):
            gather_wait(b)

            @pl.when(c + 1 < n_chunks)
            def _():
                in_wait(1 - b)

                @pl.when(c > 0)
                def _():
                    out_wait(1 - b)
                gather_start(1 - b)

            compute(b)
            out_start(c, b)

            @pl.when(c + 2 < n_chunks)
            def _():
                in_start(c + 2, b)

        in_start(0, 0)
        in_start(1, 1)
        in_wait(0)
        gather_start(0)

        def pair_body(j, carry):
            half(2 * j, 0)
            half(2 * j + 1, 1)
            return carry
        lax.fori_loop(0, pairs, pair_body, 0)

        out_wait(0)
        out_wait(1)

    return k


def kernel(location, region_id, coord_W, coord_b, region_table):
    B, L, _ = location.shape
    N = B * L
    V = region_table.shape[0]

    loc_flat = location.reshape(2 * N)
    idx2d = region_id.reshape(N // GW, GW)
    tab_pad = jnp.concatenate(
        [jnp.zeros((V, 6), jnp.float32), region_table], axis=1)

    a = coord_W[:, 0] * (1.0 / (LAT_MAX - LAT_MIN))
    b_ = coord_W[:, 1] * (1.0 / (LON_MAX - LON_MIN))
    c_ = (coord_b
          + coord_W[:, 0] * (-LAT_MIN / (LAT_MAX - LAT_MIN))
          + coord_W[:, 1] * (-LON_MIN / (LON_MAX - LON_MIN)))
    consts = jnp.stack([a, b_, c_], axis=1).reshape(18)
    consts16 = jnp.broadcast_to(consts[:, None], (18, 16))

    out = _build(N)(tab_pad, idx2d, loc_flat, consts16)
    return out.reshape(B, L, 16)


# speedup vs baseline: 2.7613x; 2.7613x over previous
import functools

import jax
import jax.numpy as jnp
from jax import lax
from jax.experimental import pallas as pl
from jax.experimental.pallas import tpu as pltpu
from jax.experimental.pallas import tpu_sc as plsc

LAT_MIN, LAT_MAX = -90.0, 90.0
LON_MIN, LON_MAX = -180.0, 180.0

NC = 2
NS = 16
NW = NC * NS
T = 1024
GW = 128
GPC = T // GW


@functools.lru_cache(maxsize=None)
def _build(N):
    R = N // NW
    n_chunks = R // T
    pairs = n_chunks // 2

    mesh = plsc.VectorSubcoreMesh(core_axis_name="c", subcore_axis_name="s")

    @functools.partial(
        pl.kernel,
        mesh=mesh,
        out_type=jax.ShapeDtypeStruct((N, 16), jnp.float32),
        compiler_params=pltpu.CompilerParams(
            needs_layout_passes=False, use_tc_tiling_on_sc=False),
        scratch_types=[
            pltpu.VMEM((2, GPC, GW), jnp.int32),
            pltpu.VMEM((2, T), jnp.float32),
            pltpu.VMEM((2, T), jnp.float32),
            pltpu.VMEM((2, T, 16), jnp.float32),
            pltpu.VMEM((18, 16), jnp.float32),
            [pltpu.SemaphoreType.DMA] * 6,
        ],
    )
    def k(tab_hbm, idx_hbm, lat_hbm, lon_hbm, const_hbm, out_hbm,
          idx_v, lat_v, lon_v, out_v, const_v, sems):
        isem, gsem, osem = sems[0:2], sems[2:4], sems[4:6]
        wid = lax.axis_index("s") * NC + lax.axis_index("c")
        base0 = wid * R
        ibase0 = wid * (R // GW)
        pltpu.sync_copy(const_hbm, const_v)
        iota = lax.iota(jnp.int32, 16)
        ca = [const_v[3 * c] for c in range(6)]
        cb = [const_v[3 * c + 1] for c in range(6)]
        cc = [const_v[3 * c + 2] for c in range(6)]
        cols = [jnp.full((16,), c, jnp.int32) for c in range(6)]

        def in_start(c, b):
            pltpu.async_copy(idx_hbm.at[pl.ds(ibase0 + c * GPC, GPC)],
                             idx_v.at[b], isem[b])
            pltpu.async_copy(lat_hbm.at[pl.ds(base0 + c * T, T)],
                             lat_v.at[b], isem[b])
            pltpu.async_copy(lon_hbm.at[pl.ds(base0 + c * T, T)],
                             lon_v.at[b], isem[b])

        def in_wait(b):
            pltpu.make_async_copy(idx_hbm.at[pl.ds(ibase0, GPC)],
                                  idx_v.at[b], isem[b]).wait()
            pltpu.make_async_copy(lat_hbm.at[pl.ds(base0, T)],
                                  lat_v.at[b], isem[b]).wait()
            pltpu.make_async_copy(lon_hbm.at[pl.ds(base0, T)],
                                  lon_v.at[b], isem[b]).wait()

        def gather_start(b):
            for j in range(GPC):
                pltpu.async_copy(tab_hbm.at[idx_v.at[b, j]],
                                 out_v.at[b, pl.ds(j * GW, GW)], gsem[b])

        def gather_wait(b):
            for j in range(GPC):
                pltpu.make_async_copy(tab_hbm.at[idx_v.at[b, j]],
                                      out_v.at[b, pl.ds(j * GW, GW)],
                                      gsem[b]).wait()

        def out_start(c, b):
            pltpu.async_copy(out_v.at[b],
                             out_hbm.at[pl.ds(base0 + c * T, T)], osem[b])

        def out_wait(b):
            pltpu.make_async_copy(out_v.at[b],
                                  out_hbm.at[pl.ds(base0, T)], osem[b]).wait()

        def compute(b):
            def group_body(g, carry):
                lat = lat_v[b, pl.ds(g * 16, 16)]
                lon = lon_v[b, pl.ds(g * 16, 16)]
                rows = g * 16 + iota
                for c in range(6):
                    vals = lat * ca[c] + lon * cb[c] + cc[c]
                    plsc.store_scatter(out_v.at[b], [rows, cols[c]], vals)
                return carry
            lax.fori_loop(0, T // 16, group_body, 0)

        def half(c, b):
            gather_wait(b)

            @pl.when(c + 1 < n_chunks)
            def _():
                in_wait(1 - b)

                @pl.when(c > 0)
                def _():
                    out_wait(1 - b)
                gather_start(1 - b)

            compute(b)
            out_start(c, b)

            @pl.when(c + 2 < n_chunks)
            def _():
                in_start(c + 2, b)

        in_start(0, 0)
        in_start(1, 1)
        in_wait(0)
        gather_start(0)

        def pair_body(j, carry):
            half(2 * j, 0)
            half(2 * j + 1, 1)
            return carry
        lax.fori_loop(0, pairs, pair_body, 0)

        out_wait(0)
        out_wait(1)

    return k


def kernel(location, region_id, coord_W, coord_b, region_table):
    B, L, _ = location.shape
    N = B * L

    lat_flat = location[:, :, 0].reshape(N)
    lon_flat = location[:, :, 1].reshape(N)
    idx2d = region_id.reshape(N // GW, GW)

    one = 1.0 + 0.0 * coord_b[0]
    tab_pad = jnp.pad(region_table, ((0, 0), (6, 0))) * one

    a = coord_W[:, 0] * (1.0 / (LAT_MAX - LAT_MIN))
    b_ = coord_W[:, 1] * (1.0 / (LON_MAX - LON_MIN))
    c_ = (coord_b
          + coord_W[:, 0] * (-LAT_MIN / (LAT_MAX - LAT_MIN))
          + coord_W[:, 1] * (-LON_MIN / (LON_MAX - LON_MIN)))
    consts = jnp.stack([a, b_, c_], axis=1).reshape(18)
    consts16 = jnp.broadcast_to(consts[:, None], (18, 16))

    out = _build(N)(tab_pad, idx2d, lat_flat, lon_flat, consts16)
    return out.reshape(B, L, 16)


# speedup vs baseline: 4.6456x; 1.6824x over previous
import functools

import jax
import jax.numpy as jnp
from jax import lax
from jax.experimental import pallas as pl
from jax.experimental.pallas import tpu as pltpu
from jax.experimental.pallas import tpu_sc as plsc

LAT_MIN, LAT_MAX = -90.0, 90.0
LON_MIN, LON_MAX = -180.0, 180.0

NC = 2
NS = 16
NW = NC * NS
GW = 128


@functools.lru_cache(maxsize=None)
def _build(B, L):
    CW = B // NW
    GPC = CW // GW
    n_chunks = L
    pairs = n_chunks // 2

    mesh = plsc.VectorSubcoreMesh(core_axis_name="c", subcore_axis_name="s")

    @functools.partial(
        pl.kernel,
        mesh=mesh,
        out_type=jax.ShapeDtypeStruct((L * 16, B), jnp.float32),
        compiler_params=pltpu.CompilerParams(
            needs_layout_passes=False, use_tc_tiling_on_sc=False),
        scratch_types=[
            pltpu.VMEM((2, GPC, GW), jnp.int32),
            pltpu.VMEM((2, CW), jnp.float32),
            pltpu.VMEM((2, CW), jnp.float32),
            pltpu.VMEM((2, CW, 16), jnp.float32),
            pltpu.VMEM((2, 16, CW), jnp.float32),
            pltpu.VMEM((18, 16), jnp.float32),
            [pltpu.SemaphoreType.DMA] * 6,
        ],
    )
    def k(tab_hbm, idx_hbm, lat_hbm, lon_hbm, const_hbm, out_hbm,
          idx_v, lat_v, lon_v, g_v, x_v, const_v, sems):
        isem, gsem, osem = sems[0:2], sems[2:4], sems[4:6]
        wid = lax.axis_index("s") * NC + lax.axis_index("c")
        col0 = wid * CW
        pltpu.sync_copy(const_hbm, const_v)
        iota = lax.iota(jnp.int32, 16)
        ca = [const_v[3 * c] for c in range(6)]
        cb = [const_v[3 * c + 1] for c in range(6)]
        cc = [const_v[3 * c + 2] for c in range(6)]
        mask6 = iota >= 6

        def in_start(c, b):
            for j in range(GPC):
                pltpu.async_copy(idx_hbm.at[c, pl.ds(col0 + j * GW, GW)],
                                 idx_v.at[b, j], isem[b])
            pltpu.async_copy(lat_hbm.at[c, pl.ds(col0, CW)],
                             lat_v.at[b], isem[b])
            pltpu.async_copy(lon_hbm.at[c, pl.ds(col0, CW)],
                             lon_v.at[b], isem[b])

        def in_wait(b):
            for j in range(GPC):
                pltpu.make_async_copy(idx_hbm.at[0, pl.ds(col0, GW)],
                                      idx_v.at[b, j], isem[b]).wait()
            pltpu.make_async_copy(lat_hbm.at[0, pl.ds(col0, CW)],
                                  lat_v.at[b], isem[b]).wait()
            pltpu.make_async_copy(lon_hbm.at[0, pl.ds(col0, CW)],
                                  lon_v.at[b], isem[b]).wait()

        def gather_start(b):
            for j in range(GPC):
                pltpu.async_copy(tab_hbm.at[idx_v.at[b, j]],
                                 g_v.at[b, pl.ds(j * GW, GW)], gsem[b])

        def gather_wait(b):
            for j in range(GPC):
                pltpu.make_async_copy(tab_hbm.at[idx_v.at[b, j]],
                                      g_v.at[b, pl.ds(j * GW, GW)],
                                      gsem[b]).wait()

        def out_start(c, b):
            pltpu.async_copy(x_v.at[b],
                             out_hbm.at[pl.ds(c * 16, 16), pl.ds(col0, CW)],
                             osem[b])

        def out_wait(b):
            pltpu.make_async_copy(x_v.at[b],
                                  out_hbm.at[pl.ds(0, 16), pl.ds(col0, CW)],
                                  osem[b]).wait()

        def coords(b):
            def group_body(g, carry):
                lat = lat_v[b, pl.ds(g * 16, 16)]
                lon = lon_v[b, pl.ds(g * 16, 16)]
                for c in range(6):
                    x_v[b, c, pl.ds(g * 16, 16)] = (
                        lat * ca[c] + lon * cb[c] + cc[c])
                return carry
            lax.fori_loop(0, CW // 16, group_body, 0)

        def embed(b):
            def tok_body(t, carry):
                row = g_v[b, t]
                tcol = iota * 0 + t
                plsc.store_scatter(x_v.at[b], [iota, tcol], row, mask=mask6)
                return carry
            lax.fori_loop(0, CW, tok_body, 0)

        def half(c, b):
            gather_wait(b)
            embed(b)
            out_start(c, b)

            @pl.when(c + 1 < n_chunks)
            def _():
                in_wait(1 - b)

                @pl.when(c > 0)
                def _():
                    out_wait(1 - b)
                gather_start(1 - b)
                coords(1 - b)

            @pl.when(c + 2 < n_chunks)
            def _():
                in_start(c + 2, b)

        in_start(0, 0)
        in_start(1, 1)
        in_wait(0)
        gather_start(0)
        coords(0)

        def pair_body(j, carry):
            half(2 * j, 0)
            half(2 * j + 1, 1)
            return carry
        lax.fori_loop(0, pairs, pair_body, 0)

        out_wait(0)
        out_wait(1)

    return k


def kernel(location, region_id, coord_W, coord_b, region_table):
    B, L, _ = location.shape
    V = region_table.shape[0]

    ridT = region_id.T
    latT = location[:, :, 0].T
    lonT = location[:, :, 1].T

    one = 1.0 + 0.0 * coord_b[0]
    tab_pad = jnp.pad(region_table, ((0, 0), (6, 0))) * one

    a = coord_W[:, 0] * (1.0 / (LAT_MAX - LAT_MIN))
    b_ = coord_W[:, 1] * (1.0 / (LON_MAX - LON_MIN))
    c_ = (coord_b
          + coord_W[:, 0] * (-LAT_MIN / (LAT_MAX - LAT_MIN))
          + coord_W[:, 1] * (-LON_MIN / (LON_MAX - LON_MIN)))
    consts = jnp.stack([a, b_, c_], axis=1).reshape(18)
    consts16 = jnp.broadcast_to(consts[:, None], (18, 16))

    out = _build(B, L)(tab_pad, ridT, latT, lonT, consts16)
    return out.reshape(L, 16, B).transpose(2, 0, 1)


# speedup vs baseline: 5.2526x; 1.1307x over previous
import functools

import jax
import jax.numpy as jnp
from jax import lax
from jax.experimental import pallas as pl
from jax.experimental.pallas import tpu as pltpu
from jax.experimental.pallas import tpu_sc as plsc

LAT_MIN, LAT_MAX = -90.0, 90.0
LON_MIN, LON_MAX = -180.0, 180.0

NC = 2
NS = 16
NW = NC * NS
GW = 128


@functools.lru_cache(maxsize=None)
def _build(B, L):
    CW = B // NW
    GPC = CW // GW
    n_chunks = L
    pairs = n_chunks // 2

    mesh = plsc.VectorSubcoreMesh(core_axis_name="c", subcore_axis_name="s")

    @functools.partial(
        pl.kernel,
        mesh=mesh,
        out_type=jax.ShapeDtypeStruct((L * 16, B), jnp.float32),
        compiler_params=pltpu.CompilerParams(
            needs_layout_passes=False, use_tc_tiling_on_sc=False),
        scratch_types=[
            pltpu.VMEM((2, GPC, GW), jnp.int32),
            pltpu.VMEM((2, CW), jnp.float32),
            pltpu.VMEM((2, CW), jnp.float32),
            pltpu.VMEM((2, CW, 16), jnp.float32),
            pltpu.VMEM((2, 16, CW), jnp.float32),
            pltpu.VMEM((18, 16), jnp.float32),
            [pltpu.SemaphoreType.DMA] * 6,
        ],
    )
    def k(tab_hbm, idx_hbm, lat_hbm, lon_hbm, const_hbm, out_hbm,
          idx_v, lat_v, lon_v, g_v, x_v, const_v, sems):
        isem, gsem, osem = sems[0:2], sems[2:4], sems[4:6]
        wid = lax.axis_index("s") * NC + lax.axis_index("c")
        col0 = wid * CW
        pltpu.sync_copy(const_hbm, const_v)
        iota = lax.iota(jnp.int32, 16)
        ca = [const_v[3 * c] for c in range(6)]
        cb = [const_v[3 * c + 1] for c in range(6)]
        cc = [const_v[3 * c + 2] for c in range(6)]
        mask6 = iota >= 6

        def in_start(c, b):
            for j in range(GPC):
                pltpu.async_copy(idx_hbm.at[c, pl.ds(col0 + j * GW, GW)],
                                 idx_v.at[b, j], isem[b])
            pltpu.async_copy(lat_hbm.at[c, pl.ds(col0, CW)],
                             lat_v.at[b], isem[b])
            pltpu.async_copy(lon_hbm.at[c, pl.ds(col0, CW)],
                             lon_v.at[b], isem[b])

        def in_wait(b):
            for j in range(GPC):
                pltpu.make_async_copy(idx_hbm.at[0, pl.ds(col0, GW)],
                                      idx_v.at[b, j], isem[b]).wait()
            pltpu.make_async_copy(lat_hbm.at[0, pl.ds(col0, CW)],
                                  lat_v.at[b], isem[b]).wait()
            pltpu.make_async_copy(lon_hbm.at[0, pl.ds(col0, CW)],
                                  lon_v.at[b], isem[b]).wait()

        def gather_start(b):
            for j in range(GPC):
                pltpu.async_copy(tab_hbm.at[idx_v.at[b, j]],
                                 g_v.at[b, pl.ds(j * GW, GW)], gsem[b])

        def gather_wait(b):
            for j in range(GPC):
                pltpu.make_async_copy(tab_hbm.at[idx_v.at[b, j]],
                                      g_v.at[b, pl.ds(j * GW, GW)],
                                      gsem[b]).wait()

        def out_start(c, b):
            pltpu.async_copy(x_v.at[b],
                             out_hbm.at[pl.ds(c * 16, 16), pl.ds(col0, CW)],
                             osem[b])

        def out_wait(b):
            pltpu.make_async_copy(x_v.at[b],
                                  out_hbm.at[pl.ds(0, 16), pl.ds(col0, CW)],
                                  osem[b]).wait()

        def coords(b):
            def group_body(g, carry):
                lat = lat_v[b, pl.ds(g * 16, 16)]
                lon = lon_v[b, pl.ds(g * 16, 16)]
                for c in range(6):
                    x_v[b, c, pl.ds(g * 16, 16)] = (
                        lat * ca[c] + lon * cb[c] + cc[c])
                return carry
            lax.fori_loop(0, CW // 16, group_body, 0)

        def embed(b):
            jcols = [jnp.full((16,), j, jnp.int32) for j in range(6, 16)]

            def group_body(g, carry):
                tvec = g * 16 + iota
                for j in range(6, 16):
                    vals = plsc.load_gather(g_v.at[b], [tvec, jcols[j - 6]])
                    x_v[b, j, pl.ds(g * 16, 16)] = vals
                return carry
            lax.fori_loop(0, CW // 16, group_body, 0)

        def half(c, b):
            gather_wait(b)
            embed(b)
            out_start(c, b)

            @pl.when(c + 1 < n_chunks)
            def _():
                in_wait(1 - b)

                @pl.when(c > 0)
                def _():
                    out_wait(1 - b)
                gather_start(1 - b)
                coords(1 - b)

            @pl.when(c + 2 < n_chunks)
            def _():
                in_start(c + 2, b)

        in_start(0, 0)
        in_start(1, 1)
        in_wait(0)
        gather_start(0)
        coords(0)

        def pair_body(j, carry):
            half(2 * j, 0)
            half(2 * j + 1, 1)
            return carry
        lax.fori_loop(0, pairs, pair_body, 0)

        out_wait(0)
        out_wait(1)

    return k


def kernel(location, region_id, coord_W, coord_b, region_table):
    B, L, _ = location.shape
    V = region_table.shape[0]

    ridT = region_id.T
    latT = location[:, :, 0].T
    lonT = location[:, :, 1].T

    one = 1.0 + 0.0 * coord_b[0]
    tab_pad = jnp.pad(region_table, ((0, 0), (6, 0))) * one

    a = coord_W[:, 0] * (1.0 / (LAT_MAX - LAT_MIN))
    b_ = coord_W[:, 1] * (1.0 / (LON_MAX - LON_MIN))
    c_ = (coord_b
          + coord_W[:, 0] * (-LAT_MIN / (LAT_MAX - LAT_MIN))
          + coord_W[:, 1] * (-LON_MIN / (LON_MAX - LON_MIN)))
    consts = jnp.stack([a, b_, c_], axis=1).reshape(18)
    consts16 = jnp.broadcast_to(consts[:, None], (18, 16))

    out = _build(B, L)(tab_pad, ridT, latT, lonT, consts16)
    return out.reshape(L, 16, B).transpose(2, 0, 1)


# speedup vs baseline: 5.8744x; 1.1184x over previous
import functools

import jax
import jax.numpy as jnp
from jax import lax
from jax.experimental import pallas as pl
from jax.experimental.pallas import tpu as pltpu
from jax.experimental.pallas import tpu_sc as plsc

LAT_MIN, LAT_MAX = -90.0, 90.0
LON_MIN, LON_MAX = -180.0, 180.0

NC = 2
NS = 16
NW = NC * NS
GW = 128


@functools.lru_cache(maxsize=None)
def _build(B, L):
    CW = B // NW
    GPC = CW // GW
    n_chunks = L
    pairs = n_chunks // 2

    mesh = plsc.VectorSubcoreMesh(core_axis_name="c", subcore_axis_name="s")

    @functools.partial(
        pl.kernel,
        mesh=mesh,
        out_type=jax.ShapeDtypeStruct((L * 16, B), jnp.float32),
        compiler_params=pltpu.CompilerParams(
            needs_layout_passes=False, use_tc_tiling_on_sc=False),
        scratch_types=[
            pltpu.VMEM((2, GPC, GW), jnp.int32),
            pltpu.VMEM((2, CW), jnp.float32),
            pltpu.VMEM((2, CW), jnp.float32),
            pltpu.VMEM((2, CW, 16), jnp.float32),
            pltpu.VMEM((2, 16, CW), jnp.float32),
            pltpu.VMEM((18, 16), jnp.float32),
            [pltpu.SemaphoreType.DMA] * 6,
        ],
    )
    def k(tab_hbm, idx_hbm, lat_hbm, lon_hbm, const_hbm, out_hbm,
          idx_v, lat_v, lon_v, g_v, x_v, const_v, sems):
        isem, gsem, osem = sems[0:2], sems[2:4], sems[4:6]
        wid = lax.axis_index("s") * NC + lax.axis_index("c")
        col0 = wid * CW
        pltpu.sync_copy(const_hbm, const_v)
        iota = lax.iota(jnp.int32, 16)
        ca = [const_v[3 * c] for c in range(6)]
        cb = [const_v[3 * c + 1] for c in range(6)]
        cc = [const_v[3 * c + 2] for c in range(6)]
        mask6 = iota >= 6

        def in_start(c, b):
            for j in range(GPC):
                pltpu.async_copy(idx_hbm.at[c, pl.ds(col0 + j * GW, GW)],
                                 idx_v.at[b, j], isem[b])
            pltpu.async_copy(lat_hbm.at[c, pl.ds(col0, CW)],
                             lat_v.at[b], isem[b])
            pltpu.async_copy(lon_hbm.at[c, pl.ds(col0, CW)],
                             lon_v.at[b], isem[b])

        def in_wait(b):
            for j in range(GPC):
                pltpu.make_async_copy(idx_hbm.at[0, pl.ds(col0, GW)],
                                      idx_v.at[b, j], isem[b]).wait()
            pltpu.make_async_copy(lat_hbm.at[0, pl.ds(col0, CW)],
                                  lat_v.at[b], isem[b]).wait()
            pltpu.make_async_copy(lon_hbm.at[0, pl.ds(col0, CW)],
                                  lon_v.at[b], isem[b]).wait()

        def gather_start(b):
            for j in range(GPC):
                pltpu.async_copy(tab_hbm.at[idx_v.at[b, j]],
                                 g_v.at[b, pl.ds(j * GW, GW)], gsem[b])

        def gather_wait(b):
            for j in range(GPC):
                pltpu.make_async_copy(tab_hbm.at[idx_v.at[b, j]],
                                      g_v.at[b, pl.ds(j * GW, GW)],
                                      gsem[b]).wait()

        def out_start(c, b):
            pltpu.async_copy(x_v.at[b],
                             out_hbm.at[pl.ds(c * 16, 16), pl.ds(col0, CW)],
                             osem[b])

        def out_wait(b):
            pltpu.make_async_copy(x_v.at[b],
                                  out_hbm.at[pl.ds(0, 16), pl.ds(col0, CW)],
                                  osem[b]).wait()

        def coords(b):
            def group_body(g, carry):
                lat = lat_v[b, pl.ds(g * 16, 16)]
                lon = lon_v[b, pl.ds(g * 16, 16)]
                for c in range(6):
                    x_v[b, c, pl.ds(g * 16, 16)] = (
                        lat * ca[c] + lon * cb[c] + cc[c])
                return carry
            lax.fori_loop(0, CW // 16, group_body, 0)

        def embed(b):
            jcols = [jnp.full((16,), j, jnp.int32) for j in range(6, 16)]

            def group_body(g, carry):
                tvec = g * 16 + iota
                for j in range(6, 16):
                    vals = plsc.load_gather(g_v.at[b], [tvec, jcols[j - 6]])
                    x_v[b, j, pl.ds(g * 16, 16)] = vals
                return carry
            lax.fori_loop(0, CW // 16, group_body, 0)

        def half(c, b):
            gather_wait(b)

            @pl.when(c + 1 < n_chunks)
            def _():
                in_wait(1 - b)
                gather_start(1 - b)

            embed(b)
            out_start(c, b)

            @pl.when(c + 1 < n_chunks)
            def _():
                @pl.when(c > 0)
                def _():
                    out_wait(1 - b)
                coords(1 - b)

            @pl.when(c + 2 < n_chunks)
            def _():
                in_start(c + 2, b)

        in_start(0, 0)
        in_start(1, 1)
        in_wait(0)
        gather_start(0)
        coords(0)

        def pair_body(j, carry):
            half(2 * j, 0)
            half(2 * j + 1, 1)
            return carry
        lax.fori_loop(0, pairs, pair_body, 0)

        out_wait(0)
        out_wait(1)

    return k


def kernel(location, region_id, coord_W, coord_b, region_table):
    B, L, _ = location.shape
    V = region_table.shape[0]

    ridT = region_id.T
    latT = location[:, :, 0].T
    lonT = location[:, :, 1].T

    one = 1.0 + 0.0 * coord_b[0]
    tab_pad = jnp.pad(region_table, ((0, 0), (6, 0))) * one

    a = coord_W[:, 0] * (1.0 / (LAT_MAX - LAT_MIN))
    b_ = coord_W[:, 1] * (1.0 / (LON_MAX - LON_MIN))
    c_ = (coord_b
          + coord_W[:, 0] * (-LAT_MIN / (LAT_MAX - LAT_MIN))
          + coord_W[:, 1] * (-LON_MIN / (LON_MAX - LON_MIN)))
    consts = jnp.stack([a, b_, c_], axis=1).reshape(18)
    consts16 = jnp.broadcast_to(consts[:, None], (18, 16))

    out = _build(B, L)(tab_pad, ridT, latT, lonT, consts16)
    return out.reshape(L, 16, B).transpose(2, 0, 1)
